# baseline (speedup 1.0000x reference)
import functools

import jax
import jax.numpy as jnp
from jax import lax
from jax.experimental import pallas as pl
from jax.experimental.pallas import tpu as pltpu
from jax.experimental.pallas import tpu_sc as plsc

N = 10000
E = 320000
D = 128
NW = 32
EPW = E // NW
LANES = 16

_mesh = plsc.VectorSubcoreMesh(core_axis_name="c", subcore_axis_name="s")
_sc_params = pltpu.CompilerParams(needs_layout_passes=False)


@functools.partial(
    pl.kernel,
    mesh=_mesh,
    out_type=[
        jax.ShapeDtypeStruct((N,), jnp.float32),
        jax.ShapeDtypeStruct((N,), jnp.float32),
    ],
    compiler_params=_sc_params,
    scratch_types=[
        pltpu.VMEM((EPW,), jnp.int32),
        pltpu.VMEM((EPW,), jnp.float32),
        pltpu.VMEM_SHARED((N,), jnp.float32),
    ],
)
def _deg_kernel(ei_hbm, ones_hbm, zeros_hbm, d0_hbm, d1_hbm,
                dst_v, ones_v, acc_sh):
    cid = lax.axis_index("c")
    sid = lax.axis_index("s")
    wid = cid * 16 + sid
    pltpu.sync_copy(ei_hbm.at[pl.ds(E + wid * EPW, EPW)], dst_v)
    pltpu.sync_copy(ones_hbm, ones_v)

    @pl.when(sid == 0)
    def _():
        pltpu.sync_copy(zeros_hbm, acc_sh)

    plsc.subcore_barrier()
    pltpu.sync_copy(ones_v, acc_sh.at[dst_v], add=True)
    plsc.subcore_barrier()

    @pl.when(jnp.logical_and(sid == 0, cid == 0))
    def _():
        pltpu.sync_copy(acc_sh, d0_hbm)

    @pl.when(jnp.logical_and(sid == 0, cid == 1))
    def _():
        pltpu.sync_copy(acc_sh, d1_hbm)


@functools.partial(
    pl.kernel,
    mesh=_mesh,
    out_type=[
        jax.ShapeDtypeStruct((N,), jnp.float32),
        jax.ShapeDtypeStruct((N,), jnp.float32),
    ],
    compiler_params=_sc_params,
    scratch_types=[
        pltpu.VMEM((EPW,), jnp.int32),
        pltpu.VMEM((EPW,), jnp.int32),
        pltpu.VMEM((N,), jnp.float32),
        pltpu.VMEM((EPW,), jnp.float32),
        pltpu.VMEM_SHARED((N,), jnp.float32),
    ],
)
def _agg_kernel(ei_hbm, s_hbm, zeros_hbm, a0_hbm, a1_hbm,
                src_v, dst_v, s_v, sv_v, acc_sh):
    cid = lax.axis_index("c")
    sid = lax.axis_index("s")
    wid = cid * 16 + sid
    pltpu.sync_copy(ei_hbm.at[pl.ds(wid * EPW, EPW)], src_v)
    pltpu.sync_copy(ei_hbm.at[pl.ds(E + wid * EPW, EPW)], dst_v)
    pltpu.sync_copy(s_hbm, s_v)

    @pl.when(sid == 0)
    def _():
        pltpu.sync_copy(zeros_hbm, acc_sh)

    def body(i, carry):
        sv_v[pl.ds(i * LANES, LANES)] = plsc.load_gather(
            s_v, [src_v[pl.ds(i * LANES, LANES)]])
        return carry

    lax.fori_loop(0, EPW // LANES, body, 0, unroll=8)

    plsc.subcore_barrier()
    pltpu.sync_copy(sv_v, acc_sh.at[dst_v], add=True)
    plsc.subcore_barrier()

    @pl.when(jnp.logical_and(sid == 0, cid == 0))
    def _():
        pltpu.sync_copy(acc_sh, a0_hbm)

    @pl.when(jnp.logical_and(sid == 0, cid == 1))
    def _():
        pltpu.sync_copy(acc_sh, a1_hbm)


def _tc0_body(x_ref, wt_ref, xw_ref):
    xw_ref[...] = lax.dot_general(
        wt_ref[...], x_ref[...], (((1,), (1,)), ((), ())),
        preferred_element_type=jnp.float32)


def _tc1_body(xw_ref, d0_ref, d1_ref, s_ref, dinv_ref, s1_ref):
    deg = (d0_ref[...] + d1_ref[...] + 1.0).reshape(1, N)
    dinv = lax.rsqrt(deg)
    dinv_ref[...] = dinv
    s = xw_ref[...] * dinv
    s_ref[...] = s
    s1_ref[...] = s.reshape(N)


def _tc2_body(a0_ref, a1_ref, s_ref, dinv_ref, b_ref, o_ref):
    tot = (a0_ref[...] + a1_ref[...]).reshape(1, N) + s_ref[...]
    o_ref[...] = jnp.maximum(dinv_ref[...] * tot + b_ref[...], 0.0)


def kernel(x, edge_index, W, b):
    ei = edge_index.astype(jnp.int32).reshape(2 * E)
    wt = W.reshape(1, D)
    b2 = b.reshape(1, 1)
    ones_e = jnp.ones((EPW,), jnp.float32)
    zeros_n = jnp.zeros((N,), jnp.float32)

    xw_row = pl.pallas_call(
        _tc0_body,
        out_shape=jax.ShapeDtypeStruct((1, N), jnp.float32),
    )(x, wt)

    d0, d1 = _deg_kernel(ei, ones_e, zeros_n)

    s_row, dinv_row, s1d = pl.pallas_call(
        _tc1_body,
        out_shape=[
            jax.ShapeDtypeStruct((1, N), jnp.float32),
            jax.ShapeDtypeStruct((1, N), jnp.float32),
            jax.ShapeDtypeStruct((N,), jnp.float32),
        ],
    )(xw_row, d0, d1)

    a0, a1 = _agg_kernel(ei, s1d, zeros_n)

    out_row = pl.pallas_call(
        _tc2_body,
        out_shape=jax.ShapeDtypeStruct((1, N), jnp.float32),
    )(a0, a1, s_row, dinv_row, b2)

    return out_row.reshape(N, 1)


# speedup vs baseline: 1.1792x; 1.1792x over previous
import functools

import jax
import jax.numpy as jnp
from jax import lax
from jax.experimental import pallas as pl
from jax.experimental.pallas import tpu as pltpu
from jax.experimental.pallas import tpu_sc as plsc

N = 10000
E = 320000
D = 128
NW = 32
EPW = E // NW
LANES = 16
TILES = E // 128
NT_HI = 79
NT_LO = 78

_mesh = plsc.VectorSubcoreMesh(core_axis_name="c", subcore_axis_name="s")
_sc_params = pltpu.CompilerParams(needs_layout_passes=False)


@functools.partial(
    pl.kernel,
    mesh=_mesh,
    out_type=jax.ShapeDtypeStruct((NW, N), jnp.float32),
    compiler_params=_sc_params,
    scratch_types=[
        pltpu.VMEM((NT_HI, 2, 128), jnp.int32),
        pltpu.VMEM((N,), jnp.float32),
    ],
)
def _deg_kernel(ei_hbm, out_hbm, ei_v, acc_v):
    wid = lax.axis_index("c") * 16 + lax.axis_index("s")
    t0 = NT_LO * wid + jnp.minimum(wid, 4)

    zeros = jnp.zeros((LANES,), jnp.float32)

    def init(i, carry):
        acc_v[pl.ds(i * LANES, LANES)] = zeros
        return carry

    lax.fori_loop(0, N // LANES, init, 0, unroll=8)

    ones = jnp.ones((LANES,), jnp.float32)

    def run(nt):
        pltpu.sync_copy(ei_hbm.at[pl.ds(t0, nt)], ei_v.at[pl.ds(0, nt)])

        def body(t, carry):
            for k in range(8):
                dv = ei_v[t, 1, pl.ds(k * LANES, LANES)]
                plsc.addupdate_scatter(acc_v, [dv], ones)
            return carry

        lax.fori_loop(0, nt, body, 0)

    @pl.when(wid < 4)
    def _():
        run(NT_HI)

    @pl.when(wid >= 4)
    def _():
        run(NT_LO)

    pltpu.sync_copy(acc_v, out_hbm.at[wid])


@functools.partial(
    pl.kernel,
    mesh=_mesh,
    out_type=jax.ShapeDtypeStruct((NW, N), jnp.float32),
    compiler_params=_sc_params,
    scratch_types=[
        pltpu.VMEM((NT_HI, 2, 128), jnp.int32),
        pltpu.VMEM((N,), jnp.float32),
        pltpu.VMEM((N,), jnp.float32),
    ],
)
def _agg_kernel(ei_hbm, s_hbm, out_hbm, ei_v, s_v, acc_v):
    wid = lax.axis_index("c") * 16 + lax.axis_index("s")
    t0 = NT_LO * wid + jnp.minimum(wid, 4)
    pltpu.sync_copy(s_hbm, s_v)

    zeros = jnp.zeros((LANES,), jnp.float32)

    def init(i, carry):
        acc_v[pl.ds(i * LANES, LANES)] = zeros
        return carry

    lax.fori_loop(0, N // LANES, init, 0, unroll=8)

    def run(nt):
        pltpu.sync_copy(ei_hbm.at[pl.ds(t0, nt)], ei_v.at[pl.ds(0, nt)])

        def body(t, carry):
            for k in range(8):
                sv = plsc.load_gather(s_v, [ei_v[t, 0, pl.ds(k * LANES, LANES)]])
                plsc.addupdate_scatter(
                    acc_v, [ei_v[t, 1, pl.ds(k * LANES, LANES)]], sv)
            return carry

        lax.fori_loop(0, nt, body, 0)

    @pl.when(wid < 4)
    def _():
        run(NT_HI)

    @pl.when(wid >= 4)
    def _():
        run(NT_LO)

    pltpu.sync_copy(acc_v, out_hbm.at[wid])


def _tc0_body(x_ref, wt_ref, xw_ref):
    xw_ref[...] = lax.dot_general(
        wt_ref[...], x_ref[...], (((1,), (1,)), ((), ())),
        preferred_element_type=jnp.float32)


def _tc1_body(xw_ref, degp_ref, s_ref, dinv_ref, s1_ref):
    deg = jnp.sum(degp_ref[...], axis=0, keepdims=True) + 1.0
    dinv = lax.rsqrt(deg)
    dinv_ref[...] = dinv
    s = xw_ref[...] * dinv
    s_ref[...] = s
    s1_ref[...] = s.reshape(N)


def _tc2_body(accp_ref, s_ref, dinv_ref, b_ref, o_ref):
    tot = jnp.sum(accp_ref[...], axis=0, keepdims=True) + s_ref[...]
    o_ref[...] = jnp.maximum(dinv_ref[...] * tot + b_ref[...], 0.0)


def kernel(x, edge_index, W, b):
    ei = edge_index.astype(jnp.int32).reshape(2, TILES, 128).transpose(1, 0, 2)
    wt = W.reshape(1, D)
    b2 = b.reshape(1, 1)

    xw_row = pl.pallas_call(
        _tc0_body,
        out_shape=jax.ShapeDtypeStruct((1, N), jnp.float32),
    )(x, wt)

    degp = _deg_kernel(ei)

    s_row, dinv_row, s1d = pl.pallas_call(
        _tc1_body,
        out_shape=[
            jax.ShapeDtypeStruct((1, N), jnp.float32),
            jax.ShapeDtypeStruct((1, N), jnp.float32),
            jax.ShapeDtypeStruct((N,), jnp.float32),
        ],
    )(xw_row, degp)

    accp = _agg_kernel(ei, s1d)

    out_row = pl.pallas_call(
        _tc2_body,
        out_shape=jax.ShapeDtypeStruct((1, N), jnp.float32),
    )(accp, s_row, dinv_row, b2)

    return out_row.reshape(N, 1)


# speedup vs baseline: 1.2207x; 1.0352x over previous
import functools

import jax
import jax.numpy as jnp
from jax import lax
from jax.experimental import pallas as pl
from jax.experimental.pallas import tpu as pltpu
from jax.experimental.pallas import tpu_sc as plsc

N = 10000
E = 320000
D = 128
NW = 32
EPW = E // NW
LANES = 16
TILES = E // 128
NT_HI = 79
NT_LO = 78

_mesh = plsc.VectorSubcoreMesh(core_axis_name="c", subcore_axis_name="s")
_sc_params = pltpu.CompilerParams(needs_layout_passes=False)


@functools.partial(
    pl.kernel,
    mesh=_mesh,
    out_type=jax.ShapeDtypeStruct((NW, N), jnp.float32),
    compiler_params=_sc_params,
    scratch_types=[
        pltpu.VMEM((NT_HI, 2, 128), jnp.int32),
        pltpu.VMEM((N,), jnp.float32),
    ],
)
def _deg_kernel(ei_hbm, out_hbm, ei_v, acc_v):
    wid = lax.axis_index("c") * 16 + lax.axis_index("s")
    t0 = NT_LO * wid + jnp.minimum(wid, 4)

    zeros = jnp.zeros((LANES,), jnp.float32)

    def init(i, carry):
        acc_v[pl.ds(i * LANES, LANES)] = zeros
        return carry

    lax.fori_loop(0, N // LANES, init, 0, unroll=8)

    ones = jnp.ones((LANES,), jnp.float32)

    def run(nt):
        pltpu.sync_copy(ei_hbm.at[pl.ds(t0, nt)], ei_v.at[pl.ds(0, nt)])

        def body(t, carry):
            for k in range(8):
                dv = ei_v[t, 1, pl.ds(k * LANES, LANES)]
                plsc.addupdate_scatter(acc_v, [dv], ones)
            return carry

        lax.fori_loop(0, nt, body, 0)

    @pl.when(wid < 4)
    def _():
        run(NT_HI)

    @pl.when(wid >= 4)
    def _():
        run(NT_LO)

    pltpu.sync_copy(acc_v, out_hbm.at[wid])


@functools.partial(
    pl.kernel,
    mesh=_mesh,
    out_type=jax.ShapeDtypeStruct((NW, N), jnp.float32),
    compiler_params=_sc_params,
    scratch_types=[
        pltpu.VMEM((NT_HI, 2, 128), jnp.int32),
        pltpu.VMEM((N,), jnp.float32),
        pltpu.VMEM((N,), jnp.float32),
    ],
)
def _agg_kernel(ei_hbm, s_hbm, out_hbm, ei_v, s_v, acc_v):
    wid = lax.axis_index("c") * 16 + lax.axis_index("s")
    t0 = NT_LO * wid + jnp.minimum(wid, 4)
    pltpu.sync_copy(s_hbm, s_v)

    zeros = jnp.zeros((LANES,), jnp.float32)

    def init(i, carry):
        acc_v[pl.ds(i * LANES, LANES)] = zeros
        return carry

    lax.fori_loop(0, N // LANES, init, 0, unroll=8)

    def run(nt):
        pltpu.sync_copy(ei_hbm.at[pl.ds(t0, nt)], ei_v.at[pl.ds(0, nt)])

        def body(t, carry):
            svs = [
                plsc.load_gather(s_v, [ei_v[t, 0, pl.ds(k * LANES, LANES)]])
                for k in range(8)
            ]
            for k in range(8):
                plsc.addupdate_scatter(
                    acc_v, [ei_v[t, 1, pl.ds(k * LANES, LANES)]], svs[k])
            return carry

        lax.fori_loop(0, nt, body, 0, unroll=2)

    @pl.when(wid < 4)
    def _():
        run(NT_HI)

    @pl.when(wid >= 4)
    def _():
        run(NT_LO)

    pltpu.sync_copy(acc_v, out_hbm.at[wid])


def _tc0_body(x_ref, wt_ref, xw_ref):
    xw_ref[...] = lax.dot_general(
        wt_ref[...], x_ref[...], (((1,), (1,)), ((), ())),
        preferred_element_type=jnp.float32)


def _tc1_body(xw_ref, degp_ref, s_ref, dinv_ref, s1_ref):
    deg = jnp.sum(degp_ref[...], axis=0, keepdims=True) + 1.0
    dinv = lax.rsqrt(deg)
    dinv_ref[...] = dinv
    s = xw_ref[...] * dinv
    s_ref[...] = s
    s1_ref[...] = s.reshape(N)


def _tc2_body(accp_ref, s_ref, dinv_ref, b_ref, o_ref):
    tot = jnp.sum(accp_ref[...], axis=0, keepdims=True) + s_ref[...]
    o_ref[...] = jnp.maximum(dinv_ref[...] * tot + b_ref[...], 0.0)


def kernel(x, edge_index, W, b):
    ei = edge_index.astype(jnp.int32).reshape(2, TILES, 128).transpose(1, 0, 2)
    wt = W.reshape(1, D)
    b2 = b.reshape(1, 1)

    xw_row = pl.pallas_call(
        _tc0_body,
        out_shape=jax.ShapeDtypeStruct((1, N), jnp.float32),
    )(x, wt)

    degp = _deg_kernel(ei)

    s_row, dinv_row, s1d = pl.pallas_call(
        _tc1_body,
        out_shape=[
            jax.ShapeDtypeStruct((1, N), jnp.float32),
            jax.ShapeDtypeStruct((1, N), jnp.float32),
            jax.ShapeDtypeStruct((N,), jnp.float32),
        ],
    )(xw_row, degp)

    accp = _agg_kernel(ei, s1d)

    out_row = pl.pallas_call(
        _tc2_body,
        out_shape=jax.ShapeDtypeStruct((1, N), jnp.float32),
    )(accp, s_row, dinv_row, b2)

    return out_row.reshape(N, 1)


# speedup vs baseline: 1.2869x; 1.0542x over previous
import functools

import jax
import jax.numpy as jnp
from jax import lax
from jax.experimental import pallas as pl
from jax.experimental.pallas import tpu as pltpu
from jax.experimental.pallas import tpu_sc as plsc

N = 10000
E = 320000
D = 128
NW = 32
EPW = E // NW
LANES = 16
TILES = E // 128
NT_HI = 79
NT_LO = 78

_mesh = plsc.VectorSubcoreMesh(core_axis_name="c", subcore_axis_name="s")
_sc_params = pltpu.CompilerParams(needs_layout_passes=False)


@functools.partial(
    pl.kernel,
    mesh=_mesh,
    out_type=jax.ShapeDtypeStruct((NW, N), jnp.float32),
    compiler_params=_sc_params,
    scratch_types=[
        pltpu.VMEM((NT_HI, 2, 128), jnp.int32),
        pltpu.VMEM((N,), jnp.float32),
    ],
)
def _deg_kernel(ei_hbm, out_hbm, ei_v, acc_v):
    wid = lax.axis_index("c") * 16 + lax.axis_index("s")
    t0 = NT_LO * wid + jnp.minimum(wid, 4)

    zeros = jnp.zeros((LANES,), jnp.float32)

    def init(i, carry):
        acc_v[pl.ds(i * LANES, LANES)] = zeros
        return carry

    lax.fori_loop(0, N // LANES, init, 0, unroll=8)

    ones = jnp.ones((LANES,), jnp.float32)

    def run(nt):
        pltpu.sync_copy(ei_hbm.at[pl.ds(t0, nt)], ei_v.at[pl.ds(0, nt)])

        def body(t, carry):
            dvs = [ei_v[t, 1, pl.ds(k * LANES, LANES)] for k in range(8)]
            for dv in dvs:
                plsc.addupdate_scatter(acc_v, [dv], ones)
            return carry

        lax.fori_loop(0, nt, body, 0, unroll=2)

    @pl.when(wid < 4)
    def _():
        run(NT_HI)

    @pl.when(wid >= 4)
    def _():
        run(NT_LO)

    pltpu.sync_copy(acc_v, out_hbm.at[wid])


@functools.partial(
    pl.kernel,
    mesh=_mesh,
    out_type=jax.ShapeDtypeStruct((NW, N), jnp.float32),
    compiler_params=_sc_params,
    scratch_types=[
        pltpu.VMEM((NT_HI, 2, 128), jnp.int32),
        pltpu.VMEM((N,), jnp.float32),
        pltpu.VMEM((N,), jnp.float32),
    ],
)
def _agg_kernel(ei_hbm, s_hbm, out_hbm, ei_v, s_v, acc_v):
    wid = lax.axis_index("c") * 16 + lax.axis_index("s")
    t0 = NT_LO * wid + jnp.minimum(wid, 4)
    pltpu.sync_copy(s_hbm, s_v)

    zeros = jnp.zeros((LANES,), jnp.float32)

    def init(i, carry):
        acc_v[pl.ds(i * LANES, LANES)] = zeros
        return carry

    lax.fori_loop(0, N // LANES, init, 0, unroll=8)

    def run(nt):
        pltpu.sync_copy(ei_hbm.at[pl.ds(t0, nt)], ei_v.at[pl.ds(0, nt)])

        def body(t, carry):
            svs = [
                plsc.load_gather(s_v, [ei_v[t, 0, pl.ds(k * LANES, LANES)]])
                for k in range(8)
            ]
            for k in range(8):
                plsc.addupdate_scatter(
                    acc_v, [ei_v[t, 1, pl.ds(k * LANES, LANES)]], svs[k])
            return carry

        lax.fori_loop(0, nt, body, 0, unroll=4)

    @pl.when(wid < 4)
    def _():
        run(NT_HI)

    @pl.when(wid >= 4)
    def _():
        run(NT_LO)

    pltpu.sync_copy(acc_v, out_hbm.at[wid])


def _tc0_body(x_ref, wt_ref, xw_ref):
    xw_ref[...] = lax.dot_general(
        wt_ref[...], x_ref[...], (((1,), (1,)), ((), ())),
        preferred_element_type=jnp.float32)


def _tc1_body(xw_ref, degp_ref, s_ref, dinv_ref, s1_ref):
    deg = jnp.sum(degp_ref[...], axis=0, keepdims=True) + 1.0
    dinv = lax.rsqrt(deg)
    dinv_ref[...] = dinv
    s = xw_ref[...] * dinv
    s_ref[...] = s
    s1_ref[...] = s.reshape(N)


def _tc2_body(accp_ref, s_ref, dinv_ref, b_ref, o_ref):
    tot = jnp.sum(accp_ref[...], axis=0, keepdims=True) + s_ref[...]
    o_ref[...] = jnp.maximum(dinv_ref[...] * tot + b_ref[...], 0.0)


def kernel(x, edge_index, W, b):
    ei = edge_index.astype(jnp.int32).reshape(2, TILES, 128).transpose(1, 0, 2)
    wt = W.reshape(1, D)
    b2 = b.reshape(1, 1)

    xw_row = pl.pallas_call(
        _tc0_body,
        out_shape=jax.ShapeDtypeStruct((1, N), jnp.float32),
    )(x, wt)

    degp = _deg_kernel(ei)

    s_row, dinv_row, s1d = pl.pallas_call(
        _tc1_body,
        out_shape=[
            jax.ShapeDtypeStruct((1, N), jnp.float32),
            jax.ShapeDtypeStruct((1, N), jnp.float32),
            jax.ShapeDtypeStruct((N,), jnp.float32),
        ],
    )(xw_row, degp)

    accp = _agg_kernel(ei, s1d)

    out_row = pl.pallas_call(
        _tc2_body,
        out_shape=jax.ShapeDtypeStruct((1, N), jnp.float32),
    )(accp, s_row, dinv_row, b2)

    return out_row.reshape(N, 1)
